# baseline (speedup 1.0000x reference)
import jax
import jax.numpy as jnp
from jax import lax
from jax.experimental import pallas as pl
from jax.experimental.pallas import tpu as pltpu
from jax.experimental.pallas import tpu_sc as plsc

DIM = 64
B_TOTAL = 16384
NC = 2
NS = 16
NW = NC * NS
BPW = B_TOTAL // NW
NCHUNK = 4
CHUNK = BPW // NCHUNK
PK = 2 * DIM
REG = 0.0001

TC_BLOCK = 2048
TC_GRID = B_TOTAL // TC_BLOCK


def _sc_gather_body(idx_hbm, tab_hbm, out_hbm, idx_v, buf, sem, osem):
    wid = lax.axis_index("s") * NC + lax.axis_index("c")
    pltpu.sync_copy(idx_hbm.at[wid], idx_v)
    cps = []
    for j in range(NCHUNK):
        dst = pl.ds(j * CHUNK, CHUNK)
        cps.append(pltpu.async_copy(tab_hbm.at[idx_v.at[j]], buf.at[dst], sem))
    for c in cps:
        c.wait()
    pltpu.async_copy(buf, out_hbm.at[pl.ds(wid * BPW, BPW)], osem).wait()


def _build_sc(vpk):
    mesh = plsc.VectorSubcoreMesh(
        core_axis_name="c", subcore_axis_name="s",
        num_cores=NC, num_subcores=NS)
    return pl.kernel(
        _sc_gather_body,
        out_type=jax.ShapeDtypeStruct((B_TOTAL, PK), jnp.float32),
        mesh=mesh,
        compiler_params=pltpu.CompilerParams(
            needs_layout_passes=False, use_tc_tiling_on_sc=True),
        scratch_types=[
            pltpu.VMEM((NCHUNK, CHUNK), jnp.int32),
            pltpu.VMEM((BPW, PK), jnp.float32),
            pltpu.SemaphoreType.DMA,
            pltpu.SemaphoreType.DMA,
        ],
    )


def _tc_body(u_ref, p_ref, n_ref, mu_ref, mp_ref, mn_ref,
             total_ref, bpr_ref, reg_ref):
    i = pl.program_id(0)

    def sel(ref, m_ref):
        x = ref[...]
        m = m_ref[...]
        return jnp.where(m > 0, x[:, DIM:], x[:, :DIM])

    u = sel(u_ref, mu_ref)
    p = sel(p_ref, mp_ref)
    n = sel(n_ref, mn_ref)
    d = jnp.sum(u * (p - n), axis=1)
    ls = jnp.sum(jnp.log(jax.nn.sigmoid(d)))
    sq = jnp.sum(u * u) + jnp.sum(p * p) + jnp.sum(n * n)

    @pl.when(i == 0)
    def _():
        bpr_ref[...] = jnp.zeros_like(bpr_ref)
        reg_ref[...] = jnp.zeros_like(reg_ref)

    bpr_ref[...] += ls
    reg_ref[...] += sq

    @pl.when(i == TC_GRID - 1)
    def _():
        b = -bpr_ref[...] / B_TOTAL
        r = REG * (reg_ref[...] / B_TOTAL)
        bpr_ref[...] = b
        reg_ref[...] = r
        total_ref[...] = b + r


def kernel(uids, pos, neg, user_emb, item_emb):
    vpk_u = user_emb.shape[0] * DIM // PK
    vpk_i = item_emb.shape[0] * DIM // PK
    upk = user_emb.reshape(vpk_u, PK)
    ipk = item_emb.reshape(vpk_i, PK)

    def prep(ids):
        pid3 = (ids // 2).reshape(NW, NCHUNK, CHUNK)
        m = (ids % 2).astype(jnp.float32).reshape(B_TOTAL, 1)
        return pid3, m

    up3, mu = prep(uids)
    pp3, mp = prep(pos)
    np3, mn = prep(neg)
    sc = _build_sc(vpk_u)
    ue = sc(up3, upk)
    pe = sc(pp3, ipk)
    ne = sc(np3, ipk)
    total, bpr, reg = pl.pallas_call(
        _tc_body,
        grid=(TC_GRID,),
        in_specs=[
            pl.BlockSpec((TC_BLOCK, PK), lambda i: (i, 0)),
            pl.BlockSpec((TC_BLOCK, PK), lambda i: (i, 0)),
            pl.BlockSpec((TC_BLOCK, PK), lambda i: (i, 0)),
            pl.BlockSpec((TC_BLOCK, 1), lambda i: (i, 0)),
            pl.BlockSpec((TC_BLOCK, 1), lambda i: (i, 0)),
            pl.BlockSpec((TC_BLOCK, 1), lambda i: (i, 0)),
        ],
        out_specs=[
            pl.BlockSpec((1, 1), lambda i: (0, 0)),
            pl.BlockSpec((1, 1), lambda i: (0, 0)),
            pl.BlockSpec((1, 1), lambda i: (0, 0)),
        ],
        out_shape=[
            jax.ShapeDtypeStruct((1, 1), jnp.float32),
            jax.ShapeDtypeStruct((1, 1), jnp.float32),
            jax.ShapeDtypeStruct((1, 1), jnp.float32),
        ],
    )(ue, pe, ne, mu, mp, mn)
    return total[0, 0], bpr[0, 0], reg[0, 0]


# speedup vs baseline: 1.0171x; 1.0171x over previous
import jax
import jax.numpy as jnp
from jax import lax
from jax.experimental import pallas as pl
from jax.experimental.pallas import tpu as pltpu
from jax.experimental.pallas import tpu_sc as plsc

DIM = 64
B_TOTAL = 16384
NC = 2
NS = 16
NW = NC * NS
BPW = B_TOTAL // NW
NCHUNK = 4
CHUNK = BPW // NCHUNK
REG = 0.0001

TC_BLOCK = 2048
TC_GRID = B_TOTAL // TC_BLOCK


def _sc_user_body(uids_hbm, uemb_hbm, uout_hbm,
                  idx_u, u_v, sem, osem):
    wid = lax.axis_index("s") * NC + lax.axis_index("c")
    pltpu.sync_copy(uids_hbm.at[wid], idx_u)
    cps = []
    for j in range(NCHUNK):
        dst = pl.ds(j * CHUNK, CHUNK)
        cps.append(pltpu.async_copy(uemb_hbm.at[idx_u.at[j]], u_v.at[dst], sem))
    for c in cps:
        c.wait()
    pltpu.async_copy(u_v, uout_hbm.at[pl.ds(wid * BPW, BPW)], osem).wait()


def _sc_item_body(pos_hbm, neg_hbm, iemb_hbm, pout_hbm, nout_hbm,
                  idx_p, idx_n, p_v, n_v, sem, osem):
    wid = lax.axis_index("s") * NC + lax.axis_index("c")
    pltpu.sync_copy(pos_hbm.at[wid], idx_p)
    pltpu.sync_copy(neg_hbm.at[wid], idx_n)
    cps = []
    for j in range(NCHUNK):
        dst = pl.ds(j * CHUNK, CHUNK)
        cps.append(pltpu.async_copy(iemb_hbm.at[idx_p.at[j]], p_v.at[dst], sem))
        cps.append(pltpu.async_copy(iemb_hbm.at[idx_n.at[j]], n_v.at[dst], sem))
    for c in cps:
        c.wait()
    out = pl.ds(wid * BPW, BPW)
    ocp = [
        pltpu.async_copy(p_v, pout_hbm.at[out], osem),
        pltpu.async_copy(n_v, nout_hbm.at[out], osem),
    ]
    for c in ocp:
        c.wait()


def _sc_mesh():
    return plsc.VectorSubcoreMesh(
        core_axis_name="c", subcore_axis_name="s",
        num_cores=NC, num_subcores=NS)


_SC_PARAMS = pltpu.CompilerParams(
    needs_layout_passes=False, use_tc_tiling_on_sc=False)


def _build_sc_user():
    return pl.kernel(
        _sc_user_body,
        out_type=jax.ShapeDtypeStruct((B_TOTAL, DIM), jnp.float32),
        mesh=_sc_mesh(),
        compiler_params=_SC_PARAMS,
        scratch_types=[
            pltpu.VMEM((NCHUNK, CHUNK), jnp.int32),
            pltpu.VMEM((BPW, DIM), jnp.float32),
            pltpu.SemaphoreType.DMA,
            pltpu.SemaphoreType.DMA,
        ],
    )


def _build_sc_item():
    return pl.kernel(
        _sc_item_body,
        out_type=[
            jax.ShapeDtypeStruct((B_TOTAL, DIM), jnp.float32),
            jax.ShapeDtypeStruct((B_TOTAL, DIM), jnp.float32),
        ],
        mesh=_sc_mesh(),
        compiler_params=_SC_PARAMS,
        scratch_types=[
            pltpu.VMEM((NCHUNK, CHUNK), jnp.int32),
            pltpu.VMEM((NCHUNK, CHUNK), jnp.int32),
            pltpu.VMEM((BPW, DIM), jnp.float32),
            pltpu.VMEM((BPW, DIM), jnp.float32),
            pltpu.SemaphoreType.DMA,
            pltpu.SemaphoreType.DMA,
        ],
    )


def _tc_body(u_ref, p_ref, n_ref, total_ref, bpr_ref, reg_ref):
    i = pl.program_id(0)
    u = u_ref[...]
    p = p_ref[...]
    n = n_ref[...]
    d = jnp.sum(u * (p - n), axis=1)
    ls = jnp.sum(jnp.log(jax.nn.sigmoid(d)))
    sq = jnp.sum(u * u) + jnp.sum(p * p) + jnp.sum(n * n)

    @pl.when(i == 0)
    def _():
        bpr_ref[...] = jnp.zeros_like(bpr_ref)
        reg_ref[...] = jnp.zeros_like(reg_ref)

    bpr_ref[...] += ls
    reg_ref[...] += sq

    @pl.when(i == TC_GRID - 1)
    def _():
        b = -bpr_ref[...] / B_TOTAL
        r = REG * (reg_ref[...] / B_TOTAL)
        bpr_ref[...] = b
        reg_ref[...] = r
        total_ref[...] = b + r


def kernel(uids, pos, neg, user_emb, item_emb):
    u3 = uids.reshape(NW, NCHUNK, CHUNK)
    p3 = pos.reshape(NW, NCHUNK, CHUNK)
    n3 = neg.reshape(NW, NCHUNK, CHUNK)
    ue = _build_sc_user()(u3, user_emb)
    pe, ne = _build_sc_item()(p3, n3, item_emb)
    total, bpr, reg = pl.pallas_call(
        _tc_body,
        grid=(TC_GRID,),
        in_specs=[
            pl.BlockSpec((TC_BLOCK, DIM), lambda i: (i, 0)),
            pl.BlockSpec((TC_BLOCK, DIM), lambda i: (i, 0)),
            pl.BlockSpec((TC_BLOCK, DIM), lambda i: (i, 0)),
        ],
        out_specs=[
            pl.BlockSpec((1, 1), lambda i: (0, 0)),
            pl.BlockSpec((1, 1), lambda i: (0, 0)),
            pl.BlockSpec((1, 1), lambda i: (0, 0)),
        ],
        out_shape=[
            jax.ShapeDtypeStruct((1, 1), jnp.float32),
            jax.ShapeDtypeStruct((1, 1), jnp.float32),
            jax.ShapeDtypeStruct((1, 1), jnp.float32),
        ],
    )(ue, pe, ne)
    return total[0, 0], bpr[0, 0], reg[0, 0]


# speedup vs baseline: 1.1981x; 1.1780x over previous
import jax
import jax.numpy as jnp
from jax import lax
from jax.experimental import pallas as pl
from jax.experimental.pallas import tpu as pltpu
from jax.experimental.pallas import tpu_sc as plsc

DIM = 64
B_TOTAL = 16384
NC = 2
NS = 16
NW = NC * NS
BPW = B_TOTAL // NW
NCHUNK = 4
CHUNK = BPW // NCHUNK
PK = 2 * DIM
REG = 0.0001

RP_LANES = 2048
RP_ROWS = RP_LANES // 2

TC_BLOCK = 2048
TC_GRID = B_TOTAL // TC_BLOCK


def _repack_body(xt_ref, out_ref):
    x = xt_ref[...]
    a = x[:, :RP_ROWS].T
    b = x[:, RP_ROWS:].T
    out_ref[...] = jnp.concatenate([a, b], axis=1)


def _repack(tabT, nrows):
    grid = (nrows + RP_LANES - 1) // RP_LANES
    return pl.pallas_call(
        _repack_body,
        grid=(grid,),
        in_specs=[pl.BlockSpec((DIM, RP_LANES), lambda i: (0, i))],
        out_specs=pl.BlockSpec((RP_ROWS, PK), lambda i: (i, 0)),
        out_shape=jax.ShapeDtypeStruct((grid * RP_ROWS, PK), jnp.float32),
    )(tabT)


def _sc_gather_body(idx_hbm, tab_hbm, out_hbm, idx_v, buf, sem, osem):
    wid = lax.axis_index("s") * NC + lax.axis_index("c")
    pltpu.sync_copy(idx_hbm.at[wid], idx_v)
    cps = []
    for j in range(NCHUNK):
        dst = pl.ds(j * CHUNK, CHUNK)
        cps.append(pltpu.async_copy(tab_hbm.at[idx_v.at[j]], buf.at[dst], sem))
    for c in cps:
        c.wait()
    pltpu.async_copy(buf, out_hbm.at[pl.ds(wid * BPW, BPW)], osem).wait()


def _build_sc():
    mesh = plsc.VectorSubcoreMesh(
        core_axis_name="c", subcore_axis_name="s",
        num_cores=NC, num_subcores=NS)
    return pl.kernel(
        _sc_gather_body,
        out_type=jax.ShapeDtypeStruct((B_TOTAL, PK), jnp.float32),
        mesh=mesh,
        compiler_params=pltpu.CompilerParams(
            needs_layout_passes=False, use_tc_tiling_on_sc=True),
        scratch_types=[
            pltpu.VMEM((NCHUNK, CHUNK), jnp.int32),
            pltpu.VMEM((BPW, PK), jnp.float32),
            pltpu.SemaphoreType.DMA,
            pltpu.SemaphoreType.DMA,
        ],
    )


def _tc_body(u_ref, p_ref, n_ref, mu_ref, mp_ref, mn_ref,
             total_ref, bpr_ref, reg_ref):
    i = pl.program_id(0)

    def sel(ref, m_ref):
        x = ref[...]
        m = m_ref[...]
        return jnp.where(m > 0, x[:, DIM:], x[:, :DIM])

    u = sel(u_ref, mu_ref)
    p = sel(p_ref, mp_ref)
    n = sel(n_ref, mn_ref)
    d = jnp.sum(u * (p - n), axis=1)
    ls = jnp.sum(jnp.log(jax.nn.sigmoid(d)))
    sq = jnp.sum(u * u) + jnp.sum(p * p) + jnp.sum(n * n)

    @pl.when(i == 0)
    def _():
        bpr_ref[...] = jnp.zeros_like(bpr_ref)
        reg_ref[...] = jnp.zeros_like(reg_ref)

    bpr_ref[...] += ls
    reg_ref[...] += sq

    @pl.when(i == TC_GRID - 1)
    def _():
        b = -bpr_ref[...] / B_TOTAL
        r = REG * (reg_ref[...] / B_TOTAL)
        bpr_ref[...] = b
        reg_ref[...] = r
        total_ref[...] = b + r


def kernel(uids, pos, neg, user_emb, item_emb):
    upk = _repack(user_emb.T, user_emb.shape[0])
    ipk = _repack(item_emb.T, item_emb.shape[0])

    def prep(ids):
        pid = (ids // RP_LANES) * RP_ROWS + (ids % RP_ROWS)
        half = (ids % RP_LANES) // RP_ROWS
        pid3 = pid.reshape(NW, NCHUNK, CHUNK)
        m = half.astype(jnp.float32).reshape(B_TOTAL, 1)
        return pid3, m

    up3, mu = prep(uids)
    pp3, mp = prep(pos)
    np3, mn = prep(neg)
    sc = _build_sc()
    ue = sc(up3, upk)
    pe = sc(pp3, ipk)
    ne = sc(np3, ipk)
    total, bpr, reg = pl.pallas_call(
        _tc_body,
        grid=(TC_GRID,),
        in_specs=[
            pl.BlockSpec((TC_BLOCK, PK), lambda i: (i, 0)),
            pl.BlockSpec((TC_BLOCK, PK), lambda i: (i, 0)),
            pl.BlockSpec((TC_BLOCK, PK), lambda i: (i, 0)),
            pl.BlockSpec((TC_BLOCK, 1), lambda i: (i, 0)),
            pl.BlockSpec((TC_BLOCK, 1), lambda i: (i, 0)),
            pl.BlockSpec((TC_BLOCK, 1), lambda i: (i, 0)),
        ],
        out_specs=[
            pl.BlockSpec((1, 1), lambda i: (0, 0)),
            pl.BlockSpec((1, 1), lambda i: (0, 0)),
            pl.BlockSpec((1, 1), lambda i: (0, 0)),
        ],
        out_shape=[
            jax.ShapeDtypeStruct((1, 1), jnp.float32),
            jax.ShapeDtypeStruct((1, 1), jnp.float32),
            jax.ShapeDtypeStruct((1, 1), jnp.float32),
        ],
    )(ue, pe, ne, mu, mp, mn)
    return total[0, 0], bpr[0, 0], reg[0, 0]
